# baseline (speedup 1.0000x reference)
import jax
import jax.numpy as jnp
from jax.experimental import pallas as pl

N = 53
H = 8
D = 32
HD = H * D
NP = 64
HN = H * NP
BS = 8

F32 = jnp.float32
BF16 = jnp.bfloat16


def _gat_body(data_ref, loading_ref, W1_ref, b1_ref,
              fc1_ref, fcal1_ref, fcar1_ref, bg1_ref,
              fc2_ref, fcal2_ref, fcar2_ref, bg2_ref,
              fc3_ref, fcal3_ref, fcar3_ref, bg3_ref,
              msum_ref, e32_ref, rep8_ref, tile_ref,
              Wl_ref, bl_ref, Wlast_ref, blast_ref,
              out_ref):
    msum = msum_ref[...]
    e32 = e32_ref[...]

    hs = [None] * BS
    for b in range(BS):
        x = data_ref[b].astype(BF16)
        h_b = jnp.dot(x, W1_ref[...], preferred_element_type=F32) + b1_ref[...]
        hs[b] = jnp.maximum(h_b, 0.0)

    layers = ((fc1_ref, fcal1_ref, fcar1_ref, bg1_ref),
              (fc2_ref, fcal2_ref, fcar2_ref, bg2_ref),
              (fc3_ref, fcal3_ref, fcar3_ref, bg3_ref))
    feats = []
    for fc_ref, fcal_ref, fcar_ref, bg_ref in layers:
        fs = []
        for b in range(BS):
            hb16 = hs[b].astype(BF16)
            ftb = jnp.dot(hb16, fc_ref[...],
                          preferred_element_type=F32).astype(BF16)
            er8 = jnp.dot(hb16, fcar_ref[...], preferred_element_type=F32)
            elblk = jax.lax.dot_general(
                fcal_ref[...], hb16, (((0,), (1,)), ((), ())),
                preferred_element_type=F32)
            elpad = jnp.pad(elblk, ((0, 0), (0, NP - N)))
            eboth = jnp.exp(jnp.concatenate([elpad, 0.2 * elpad], axis=0))
            rboth = jnp.dot(eboth.astype(BF16), tile_ref[...],
                            preferred_element_type=F32)
            rp1 = rep8_ref[...] * rboth[0:H]
            rp2 = rep8_ref[...] * rboth[H:2 * H]
            u1 = jnp.dot(jnp.exp(er8), rp1, preferred_element_type=F32)
            u2 = jnp.dot(jnp.exp(0.2 * er8), rp2, preferred_element_type=F32)
            exb = jnp.maximum(u1, u2).astype(BF16)
            den = jnp.dot(exb, msum, preferred_element_type=F32)
            screp = jnp.dot((1.0 / den).astype(BF16), e32,
                            preferred_element_type=F32)
            ftp = jnp.pad(ftb, ((0, NP - N), (0, 0)))
            ftstack = jnp.concatenate(
                [ftp * e32[hd:hd + 1, :] for hd in range(H)], axis=0)
            raw = jnp.dot(exb, ftstack, preferred_element_type=F32)
            hs[b] = jnp.maximum(raw * screp + hs[b] + bg_ref[...], 0.0)
            fs.append(jnp.sum(hs[b], axis=0, keepdims=True))
        feats.append(jnp.concatenate(fs, axis=0))

    lf = jnp.dot(loading_ref[...].astype(BF16), Wl_ref[...],
                 preferred_element_type=F32)
    lf = lf + bl_ref[...]
    lf = jnp.where(lf >= 0.0, lf, 0.01 * lf)
    lfb = lf.astype(BF16)

    f1 = feats[0].astype(BF16)
    f2 = feats[1].astype(BF16)
    f3 = feats[2].astype(BF16)
    o = jnp.dot(f1, Wlast_ref[0:HD, :], preferred_element_type=F32)
    o = o + jnp.dot(f2, Wlast_ref[HD:2 * HD, :], preferred_element_type=F32)
    o = o + jnp.dot(f3, Wlast_ref[2 * HD:3 * HD, :], preferred_element_type=F32)
    o = o + jnp.dot(lfb, Wlast_ref[3 * HD:3 * HD + 128, :],
                    preferred_element_type=F32)
    out_ref[...] = o + blast_ref[...]


def _block_diag_attn(a):
    mask = jnp.kron(jnp.eye(H, dtype=F32), jnp.ones((D, 1), dtype=F32))
    return mask * a.reshape(HD, 1)


def kernel(data, loading, edge_index, W1, b1, fcW1, al1, ar1, bg1,
           fcW2, al2, ar2, bg2, fcW3, al3, ar3, bg3, Wl, bl, Wlast, blast):
    B = data.shape[0]

    def prep_layer(fcW, al, ar):
        albd = _block_diag_attn(al)
        arbd = _block_diag_attn(ar)
        fcal = jnp.dot(fcW, albd)
        fcar = jnp.dot(fcW, arbd)
        return fcW.astype(BF16), fcal.astype(BF16), fcar.astype(BF16)

    fc1b, fcal1, fcar1 = prep_layer(fcW1, al1, ar1)
    fc2b, fcal2, fcar2 = prep_layer(fcW2, al2, ar2)
    fc3b, fcal3, fcar3 = prep_layer(fcW3, al3, ar3)

    lane_i = jnp.arange(HN) % NP
    msum = jnp.kron(jnp.eye(H, dtype=F32), jnp.ones((NP, 1), dtype=F32))
    msum = (msum * (lane_i < N).astype(F32)[:, None]).astype(BF16)
    e32 = jnp.kron(jnp.eye(H, dtype=BF16), jnp.ones((1, D), dtype=BF16))
    rep8 = jnp.kron(jnp.eye(H, dtype=F32), jnp.ones((1, NP), dtype=F32))
    tile = jnp.tile(jnp.eye(NP, dtype=BF16), (1, H))

    def fixed(shape):
        nd = len(shape)
        return pl.BlockSpec(shape, lambda i: (0,) * nd)

    out = pl.pallas_call(
        _gat_body,
        grid=(B // BS,),
        in_specs=[
            pl.BlockSpec((BS, N, 400), lambda i: (i, 0, 0)),
            pl.BlockSpec((BS, 26), lambda i: (i, 0)),
            fixed((400, HD)), fixed((1, HD)),
            fixed((HD, HD)), fixed((HD, H)), fixed((HD, H)), fixed((1, HD)),
            fixed((HD, HD)), fixed((HD, H)), fixed((HD, H)), fixed((1, HD)),
            fixed((HD, HD)), fixed((HD, H)), fixed((HD, H)), fixed((1, HD)),
            fixed((HN, H)), fixed((H, HD)), fixed((H, HN)), fixed((NP, HN)),
            fixed((26, 128)), fixed((1, 128)),
            fixed((3 * HD + 128, 10)), fixed((1, 10)),
        ],
        out_specs=pl.BlockSpec((BS, 10), lambda i: (i, 0)),
        out_shape=jax.ShapeDtypeStruct((B, 10), F32),
    )(data, loading, W1.astype(BF16), b1.reshape(1, HD),
      fc1b, fcal1, fcar1, bg1.reshape(1, HD),
      fc2b, fcal2, fcar2, bg2.reshape(1, HD),
      fc3b, fcal3, fcar3, bg3.reshape(1, HD),
      msum, e32, rep8, tile,
      Wl.astype(BF16), bl.reshape(1, 128),
      Wlast.astype(BF16), blast.reshape(1, 10))
    return out


# speedup vs baseline: 2.4569x; 2.4569x over previous
import jax
import jax.numpy as jnp
from jax.experimental import pallas as pl

N = 53
H = 8
D = 32
HD = H * D
NP = 64
HN = H * NP
BS = 8
SN = BS * NP

F32 = jnp.float32
BF16 = jnp.bfloat16


def _gat_body(data_ref, loading_ref, W1_ref, b1_ref,
              fc1_ref, fcal1_ref, fcar1_ref, bg1_ref,
              fc2_ref, fcal2_ref, fcar2_ref, bg2_ref,
              fc3_ref, fcal3_ref, fcar3_ref, bg3_ref,
              msum_ref, e32_ref, rep8_ref, colmask_ref, rowmask_ref,
              summat_ref,
              Wl_ref, bl_ref, Wlast_ref, blast_ref,
              out_ref):
    msum = msum_ref[...]
    e32 = e32_ref[...]
    rep8 = rep8_ref[...]
    colmask = colmask_ref[...]
    rowmask = rowmask_ref[...]

    hs_list = []
    for b in range(BS):
        x = data_ref[b].astype(BF16)
        h_b = jnp.dot(x, W1_ref[...], preferred_element_type=F32) + b1_ref[...]
        hs_list.append(jnp.pad(jnp.maximum(h_b, 0.0), ((0, NP - N), (0, 0))))
    HS = jnp.concatenate(hs_list, axis=0)

    layers = ((fc1_ref, fcal1_ref, fcar1_ref, bg1_ref),
              (fc2_ref, fcal2_ref, fcar2_ref, bg2_ref),
              (fc3_ref, fcal3_ref, fcar3_ref, bg3_ref))
    feats = []
    for fc_ref, fcal_ref, fcart_ref, bg_ref in layers:
        HSb = HS.astype(BF16)
        FT = jnp.dot(HSb, fc_ref[...],
                     preferred_element_type=F32).astype(BF16)
        ERT = jnp.dot(HSb, fcart_ref[...],
                      preferred_element_type=F32)
        HB2 = jnp.concatenate(
            [HSb[NP * b:NP * (b + 1)] for b in range(BS) for _ in range(2)],
            axis=0)
        EL2 = jax.lax.dot_general(
            fcal_ref[...], HB2, (((0,), (1,)), ((), ())),
            preferred_element_type=F32)
        rp1s, rp2s = [], []
        for b in range(BS):
            el2b = EL2[:, 128 * b:128 * (b + 1)]
            rp1s.append(rep8 * jnp.tile(jnp.exp(el2b), (1, HN // 128)))
            rp2s.append(rep8 * jnp.tile(jnp.exp(0.2 * el2b), (1, HN // 128)))
        RP1 = jnp.concatenate(rp1s, axis=0).astype(BF16)
        RP2 = jnp.concatenate(rp2s, axis=0).astype(BF16)
        E1 = (colmask * jnp.exp(ERT)).astype(BF16)
        E2 = (colmask * jnp.exp(0.2 * ERT)).astype(BF16)
        U1 = jnp.dot(E1, RP1, preferred_element_type=F32)
        U2 = jnp.dot(E2, RP2, preferred_element_type=F32)
        EXB = jnp.maximum(U1, U2).astype(BF16)
        DEN = jnp.dot(EXB, msum, preferred_element_type=F32)
        SCREP = jnp.dot((1.0 / DEN).astype(BF16), e32,
                        preferred_element_type=F32)
        raws = []
        for b in range(BS):
            ftb = FT[NP * b:NP * (b + 1)]
            ftstack = jnp.concatenate(
                [ftb * e32[hd:hd + 1, :] for hd in range(H)], axis=0)
            raws.append(jnp.dot(EXB[NP * b:NP * (b + 1)], ftstack,
                                preferred_element_type=F32))
        RAW = jnp.concatenate(raws, axis=0)
        HS = rowmask * jnp.maximum(RAW * SCREP + HS + bg_ref[...], 0.0)
        feats.append(jnp.dot(summat_ref[...], HS.astype(BF16),
                             preferred_element_type=F32))

    lf = jnp.dot(loading_ref[...].astype(BF16), Wl_ref[...],
                 preferred_element_type=F32)
    lf = lf + bl_ref[...]
    lf = jnp.where(lf >= 0.0, lf, 0.01 * lf)
    lfb = lf.astype(BF16)

    f1 = feats[0].astype(BF16)
    f2 = feats[1].astype(BF16)
    f3 = feats[2].astype(BF16)
    o = jnp.dot(f1, Wlast_ref[0:HD, :], preferred_element_type=F32)
    o = o + jnp.dot(f2, Wlast_ref[HD:2 * HD, :], preferred_element_type=F32)
    o = o + jnp.dot(f3, Wlast_ref[2 * HD:3 * HD, :], preferred_element_type=F32)
    o = o + jnp.dot(lfb, Wlast_ref[3 * HD:3 * HD + 128, :],
                    preferred_element_type=F32)
    out_ref[...] = o + blast_ref[...]


def _block_diag_attn(a):
    mask = jnp.kron(jnp.eye(H, dtype=F32), jnp.ones((D, 1), dtype=F32))
    return mask * a.reshape(HD, 1)


def kernel(data, loading, edge_index, W1, b1, fcW1, al1, ar1, bg1,
           fcW2, al2, ar2, bg2, fcW3, al3, ar3, bg3, Wl, bl, Wlast, blast):
    B = data.shape[0]

    def prep_layer(fcW, al, ar):
        albd = _block_diag_attn(al)
        arbd = _block_diag_attn(ar)
        fcal = jnp.dot(fcW, albd)
        fcar = jnp.dot(fcW, arbd)
        fcart = jnp.tile(fcar, (1, BS))
        return fcW.astype(BF16), fcal.astype(BF16), fcart.astype(BF16)

    fc1b, fcal1, fcar1 = prep_layer(fcW1, al1, ar1)
    fc2b, fcal2, fcar2 = prep_layer(fcW2, al2, ar2)
    fc3b, fcal3, fcar3 = prep_layer(fcW3, al3, ar3)

    lane_i = jnp.arange(HN) % NP
    msum = jnp.kron(jnp.eye(H, dtype=F32), jnp.ones((NP, 1), dtype=F32))
    msum = (msum * (lane_i < N).astype(F32)[:, None]).astype(BF16)
    e32 = jnp.kron(jnp.eye(H, dtype=BF16), jnp.ones((1, D), dtype=BF16))
    rep8 = jnp.kron(jnp.eye(H, dtype=F32), jnp.ones((1, NP), dtype=F32))
    colmask = jnp.kron(jnp.eye(BS, dtype=F32), jnp.ones((NP, H), dtype=F32))
    rowvalid = (jnp.arange(SN) % NP < N).astype(F32)
    rowmask = jnp.tile(rowvalid[:, None], (1, HD))
    summat = jnp.kron(jnp.eye(BS, dtype=F32),
                      (jnp.arange(NP) < N).astype(F32)[None, :]).astype(BF16)

    def fixed(shape):
        nd = len(shape)
        return pl.BlockSpec(shape, lambda i: (0,) * nd)

    out = pl.pallas_call(
        _gat_body,
        grid=(B // BS,),
        in_specs=[
            pl.BlockSpec((BS, N, 400), lambda i: (i, 0, 0)),
            pl.BlockSpec((BS, 26), lambda i: (i, 0)),
            fixed((400, HD)), fixed((1, HD)),
            fixed((HD, HD)), fixed((HD, H)), fixed((HD, BS * H)), fixed((1, HD)),
            fixed((HD, HD)), fixed((HD, H)), fixed((HD, BS * H)), fixed((1, HD)),
            fixed((HD, HD)), fixed((HD, H)), fixed((HD, BS * H)), fixed((1, HD)),
            fixed((HN, H)), fixed((H, HD)), fixed((H, HN)),
            fixed((SN, BS * H)), fixed((SN, HD)), fixed((BS, SN)),
            fixed((26, 128)), fixed((1, 128)),
            fixed((3 * HD + 128, 10)), fixed((1, 10)),
        ],
        out_specs=pl.BlockSpec((BS, 10), lambda i: (i, 0)),
        out_shape=jax.ShapeDtypeStruct((B, 10), F32),
    )(data, loading, W1.astype(BF16), b1.reshape(1, HD),
      fc1b, fcal1, fcar1, bg1.reshape(1, HD),
      fc2b, fcal2, fcar2, bg2.reshape(1, HD),
      fc3b, fcal3, fcar3, bg3.reshape(1, HD),
      msum, e32, rep8, colmask, rowmask, summat,
      Wl.astype(BF16), bl.reshape(1, 128),
      Wlast.astype(BF16), blast.reshape(1, 10))
    return out


# speedup vs baseline: 2.7830x; 1.1327x over previous
import jax
import jax.numpy as jnp
from jax.experimental import pallas as pl

N = 53
H = 8
D = 32
HD = H * D
NP = 64
HN = H * NP
BS = 16
SN = BS * NP

F32 = jnp.float32
BF16 = jnp.bfloat16


def _gat_body(data_ref, loading_ref, W1_ref, b1_ref,
              fc1_ref, fcal1_ref, fcar1_ref, bg1_ref,
              fc2_ref, fcal2_ref, fcar2_ref, bg2_ref,
              fc3_ref, fcal3_ref, fcar3_ref, bg3_ref,
              msum_ref, e32_ref, rep8_ref, colmask_ref, rowmask_ref,
              summat_ref,
              Wl_ref, bl_ref, Wlast_ref, blast_ref,
              out_ref):
    msum = msum_ref[...]
    e32 = e32_ref[...]
    rep8 = rep8_ref[...]
    colmask = colmask_ref[...]
    rowmask = rowmask_ref[...]

    hs_list = []
    for b in range(BS):
        x = data_ref[b].astype(BF16)
        h_b = jnp.dot(x, W1_ref[...], preferred_element_type=F32) + b1_ref[...]
        hs_list.append(jnp.pad(jnp.maximum(h_b, 0.0), ((0, NP - N), (0, 0))))
    HS = jnp.concatenate(hs_list, axis=0)

    layers = ((fc1_ref, fcal1_ref, fcar1_ref, bg1_ref),
              (fc2_ref, fcal2_ref, fcar2_ref, bg2_ref),
              (fc3_ref, fcal3_ref, fcar3_ref, bg3_ref))
    feats = []
    for fc_ref, fcal_ref, fcart_ref, bg_ref in layers:
        HSb = HS.astype(BF16)
        FT = jnp.dot(HSb, fc_ref[...],
                     preferred_element_type=F32).astype(BF16)
        ERT = jnp.dot(HSb, fcart_ref[...],
                      preferred_element_type=F32)
        HB2 = jnp.concatenate(
            [HSb[NP * b:NP * (b + 1)] for b in range(BS) for _ in range(2)],
            axis=0)
        EL2 = jax.lax.dot_general(
            fcal_ref[...], HB2, (((0,), (1,)), ((), ())),
            preferred_element_type=F32)
        rp1s, rp2s = [], []
        for b in range(BS):
            el2b = EL2[:, 128 * b:128 * (b + 1)]
            rp1s.append(rep8 * jnp.tile(jnp.exp(el2b), (1, HN // 128)))
            rp2s.append(rep8 * jnp.tile(jnp.exp(0.2 * el2b), (1, HN // 128)))
        RP1 = jnp.concatenate(rp1s, axis=0).astype(BF16)
        RP2 = jnp.concatenate(rp2s, axis=0).astype(BF16)
        E1 = (colmask * jnp.exp(ERT)).astype(BF16)
        E2 = (colmask * jnp.exp(0.2 * ERT)).astype(BF16)
        U1 = jnp.dot(E1, RP1, preferred_element_type=F32)
        U2 = jnp.dot(E2, RP2, preferred_element_type=F32)
        EXB = jnp.maximum(U1, U2).astype(BF16)
        DEN = jnp.dot(EXB, msum, preferred_element_type=F32)
        SCREP = jnp.dot((1.0 / DEN).astype(BF16), e32,
                        preferred_element_type=F32)
        raws = []
        for b in range(BS):
            ftb = FT[NP * b:NP * (b + 1)]
            ftstack = jnp.concatenate(
                [ftb * e32[hd:hd + 1, :] for hd in range(H)], axis=0)
            raws.append(jnp.dot(EXB[NP * b:NP * (b + 1)], ftstack,
                                preferred_element_type=F32))
        RAW = jnp.concatenate(raws, axis=0)
        HS = rowmask * jnp.maximum(RAW * SCREP + HS + bg_ref[...], 0.0)
        feats.append(jnp.dot(summat_ref[...], HS.astype(BF16),
                             preferred_element_type=F32))

    lf = jnp.dot(loading_ref[...].astype(BF16), Wl_ref[...],
                 preferred_element_type=F32)
    lf = lf + bl_ref[...]
    lf = jnp.where(lf >= 0.0, lf, 0.01 * lf)
    lfb = lf.astype(BF16)

    f1 = feats[0].astype(BF16)
    f2 = feats[1].astype(BF16)
    f3 = feats[2].astype(BF16)
    o = jnp.dot(f1, Wlast_ref[0:HD, :], preferred_element_type=F32)
    o = o + jnp.dot(f2, Wlast_ref[HD:2 * HD, :], preferred_element_type=F32)
    o = o + jnp.dot(f3, Wlast_ref[2 * HD:3 * HD, :], preferred_element_type=F32)
    o = o + jnp.dot(lfb, Wlast_ref[3 * HD:3 * HD + 128, :],
                    preferred_element_type=F32)
    out_ref[...] = o + blast_ref[...]


def _block_diag_attn(a):
    mask = jnp.kron(jnp.eye(H, dtype=F32), jnp.ones((D, 1), dtype=F32))
    return mask * a.reshape(HD, 1)


def kernel(data, loading, edge_index, W1, b1, fcW1, al1, ar1, bg1,
           fcW2, al2, ar2, bg2, fcW3, al3, ar3, bg3, Wl, bl, Wlast, blast):
    B = data.shape[0]

    def prep_layer(fcW, al, ar):
        albd = _block_diag_attn(al)
        arbd = _block_diag_attn(ar)
        fcal = jnp.dot(fcW, albd)
        fcar = jnp.dot(fcW, arbd)
        fcart = jnp.tile(fcar, (1, BS))
        return fcW.astype(BF16), fcal.astype(BF16), fcart.astype(BF16)

    fc1b, fcal1, fcar1 = prep_layer(fcW1, al1, ar1)
    fc2b, fcal2, fcar2 = prep_layer(fcW2, al2, ar2)
    fc3b, fcal3, fcar3 = prep_layer(fcW3, al3, ar3)

    lane_i = jnp.arange(HN) % NP
    msum = jnp.kron(jnp.eye(H, dtype=F32), jnp.ones((NP, 1), dtype=F32))
    msum = (msum * (lane_i < N).astype(F32)[:, None]).astype(BF16)
    e32 = jnp.kron(jnp.eye(H, dtype=BF16), jnp.ones((1, D), dtype=BF16))
    rep8 = jnp.kron(jnp.eye(H, dtype=F32), jnp.ones((1, NP), dtype=F32))
    colmask = jnp.kron(jnp.eye(BS, dtype=F32), jnp.ones((NP, H), dtype=F32))
    rowvalid = (jnp.arange(SN) % NP < N).astype(F32)
    rowmask = jnp.tile(rowvalid[:, None], (1, HD))
    summat = jnp.kron(jnp.eye(BS, dtype=F32),
                      (jnp.arange(NP) < N).astype(F32)[None, :]).astype(BF16)

    def fixed(shape):
        nd = len(shape)
        return pl.BlockSpec(shape, lambda i: (0,) * nd)

    out = pl.pallas_call(
        _gat_body,
        grid=(B // BS,),
        in_specs=[
            pl.BlockSpec((BS, N, 400), lambda i: (i, 0, 0)),
            pl.BlockSpec((BS, 26), lambda i: (i, 0)),
            fixed((400, HD)), fixed((1, HD)),
            fixed((HD, HD)), fixed((HD, H)), fixed((HD, BS * H)), fixed((1, HD)),
            fixed((HD, HD)), fixed((HD, H)), fixed((HD, BS * H)), fixed((1, HD)),
            fixed((HD, HD)), fixed((HD, H)), fixed((HD, BS * H)), fixed((1, HD)),
            fixed((HN, H)), fixed((H, HD)), fixed((H, HN)),
            fixed((SN, BS * H)), fixed((SN, HD)), fixed((BS, SN)),
            fixed((26, 128)), fixed((1, 128)),
            fixed((3 * HD + 128, 10)), fixed((1, 10)),
        ],
        out_specs=pl.BlockSpec((BS, 10), lambda i: (i, 0)),
        out_shape=jax.ShapeDtypeStruct((B, 10), F32),
    )(data, loading, W1.astype(BF16), b1.reshape(1, HD),
      fc1b, fcal1, fcar1, bg1.reshape(1, HD),
      fc2b, fcal2, fcar2, bg2.reshape(1, HD),
      fc3b, fcal3, fcar3, bg3.reshape(1, HD),
      msum, e32, rep8, colmask, rowmask, summat,
      Wl.astype(BF16), bl.reshape(1, 128),
      Wlast.astype(BF16), blast.reshape(1, 10))
    return out


# speedup vs baseline: 2.8847x; 1.0365x over previous
import jax
import jax.numpy as jnp
from jax.experimental import pallas as pl

N = 53
H = 8
D = 32
HD = H * D
NP = 64
HN = H * NP
BS = 32
SN = BS * NP

F32 = jnp.float32
BF16 = jnp.bfloat16


def _gat_body(data_ref, loading_ref, W1_ref, b1_ref,
              fc1_ref, fcal1_ref, fcar1_ref, bg1_ref,
              fc2_ref, fcal2_ref, fcar2_ref, bg2_ref,
              fc3_ref, fcal3_ref, fcar3_ref, bg3_ref,
              msum_ref, e32_ref, rep8_ref, colmask_ref, rowmask_ref,
              summat_ref,
              Wl_ref, bl_ref, Wlast_ref, blast_ref,
              out_ref):
    msum = msum_ref[...]
    e32 = e32_ref[...]
    rep8 = rep8_ref[...]
    colmask = colmask_ref[...]
    rowmask = rowmask_ref[...]

    hs_list = []
    for b in range(BS):
        x = data_ref[b].astype(BF16)
        h_b = jnp.dot(x, W1_ref[...], preferred_element_type=F32) + b1_ref[...]
        hs_list.append(jnp.pad(jnp.maximum(h_b, 0.0), ((0, NP - N), (0, 0))))
    HS = jnp.concatenate(hs_list, axis=0)

    layers = ((fc1_ref, fcal1_ref, fcar1_ref, bg1_ref),
              (fc2_ref, fcal2_ref, fcar2_ref, bg2_ref),
              (fc3_ref, fcal3_ref, fcar3_ref, bg3_ref))
    feats = []
    for fc_ref, fcal_ref, fcart_ref, bg_ref in layers:
        HSb = HS.astype(BF16)
        FT = jnp.dot(HSb, fc_ref[...],
                     preferred_element_type=F32).astype(BF16)
        ERT = jnp.dot(HSb, fcart_ref[...],
                      preferred_element_type=F32)
        HB2 = jnp.concatenate(
            [HSb[NP * b:NP * (b + 1)] for b in range(BS) for _ in range(2)],
            axis=0)
        EL2 = jax.lax.dot_general(
            fcal_ref[...], HB2, (((0,), (1,)), ((), ())),
            preferred_element_type=F32)
        rp1s, rp2s = [], []
        for b in range(BS):
            el2b = EL2[:, 128 * b:128 * (b + 1)]
            rp1s.append(rep8 * jnp.tile(jnp.exp(el2b), (1, HN // 128)))
            rp2s.append(rep8 * jnp.tile(jnp.exp(0.2 * el2b), (1, HN // 128)))
        RP1 = jnp.concatenate(rp1s, axis=0).astype(BF16)
        RP2 = jnp.concatenate(rp2s, axis=0).astype(BF16)
        E1 = (colmask * jnp.exp(ERT)).astype(BF16)
        E2 = (colmask * jnp.exp(0.2 * ERT)).astype(BF16)
        U1 = jnp.dot(E1, RP1, preferred_element_type=F32)
        U2 = jnp.dot(E2, RP2, preferred_element_type=F32)
        EXB = jnp.maximum(U1, U2).astype(BF16)
        DEN = jnp.dot(EXB, msum, preferred_element_type=F32)
        SCREP = jnp.dot((1.0 / DEN).astype(BF16), e32,
                        preferred_element_type=F32)
        raws = []
        for b in range(BS):
            ftb = FT[NP * b:NP * (b + 1)]
            ftstack = jnp.concatenate(
                [ftb * e32[hd:hd + 1, :] for hd in range(H)], axis=0)
            raws.append(jnp.dot(EXB[NP * b:NP * (b + 1)], ftstack,
                                preferred_element_type=F32))
        RAW = jnp.concatenate(raws, axis=0)
        HS = rowmask * jnp.maximum(RAW * SCREP + HS + bg_ref[...], 0.0)
        feats.append(jnp.dot(summat_ref[...], HS.astype(BF16),
                             preferred_element_type=F32))

    lf = jnp.dot(loading_ref[...].astype(BF16), Wl_ref[...],
                 preferred_element_type=F32)
    lf = lf + bl_ref[...]
    lf = jnp.where(lf >= 0.0, lf, 0.01 * lf)
    lfb = lf.astype(BF16)

    f1 = feats[0].astype(BF16)
    f2 = feats[1].astype(BF16)
    f3 = feats[2].astype(BF16)
    o = jnp.dot(f1, Wlast_ref[0:HD, :], preferred_element_type=F32)
    o = o + jnp.dot(f2, Wlast_ref[HD:2 * HD, :], preferred_element_type=F32)
    o = o + jnp.dot(f3, Wlast_ref[2 * HD:3 * HD, :], preferred_element_type=F32)
    o = o + jnp.dot(lfb, Wlast_ref[3 * HD:3 * HD + 128, :],
                    preferred_element_type=F32)
    out_ref[...] = o + blast_ref[...]


def _block_diag_attn(a):
    mask = jnp.kron(jnp.eye(H, dtype=F32), jnp.ones((D, 1), dtype=F32))
    return mask * a.reshape(HD, 1)


def kernel(data, loading, edge_index, W1, b1, fcW1, al1, ar1, bg1,
           fcW2, al2, ar2, bg2, fcW3, al3, ar3, bg3, Wl, bl, Wlast, blast):
    B = data.shape[0]

    def prep_layer(fcW, al, ar):
        albd = _block_diag_attn(al)
        arbd = _block_diag_attn(ar)
        fcal = jnp.dot(fcW, albd)
        fcar = jnp.dot(fcW, arbd)
        fcart = jnp.tile(fcar, (1, BS))
        return fcW.astype(BF16), fcal.astype(BF16), fcart.astype(BF16)

    fc1b, fcal1, fcar1 = prep_layer(fcW1, al1, ar1)
    fc2b, fcal2, fcar2 = prep_layer(fcW2, al2, ar2)
    fc3b, fcal3, fcar3 = prep_layer(fcW3, al3, ar3)

    lane_i = jnp.arange(HN) % NP
    msum = jnp.kron(jnp.eye(H, dtype=F32), jnp.ones((NP, 1), dtype=F32))
    msum = (msum * (lane_i < N).astype(F32)[:, None]).astype(BF16)
    e32 = jnp.kron(jnp.eye(H, dtype=BF16), jnp.ones((1, D), dtype=BF16))
    rep8 = jnp.kron(jnp.eye(H, dtype=F32), jnp.ones((1, NP), dtype=F32))
    colmask = jnp.kron(jnp.eye(BS, dtype=F32), jnp.ones((NP, H), dtype=F32))
    rowvalid = (jnp.arange(SN) % NP < N).astype(F32)
    rowmask = jnp.tile(rowvalid[:, None], (1, HD))
    summat = jnp.kron(jnp.eye(BS, dtype=F32),
                      (jnp.arange(NP) < N).astype(F32)[None, :]).astype(BF16)

    def fixed(shape):
        nd = len(shape)
        return pl.BlockSpec(shape, lambda i: (0,) * nd)

    out = pl.pallas_call(
        _gat_body,
        grid=(B // BS,),
        in_specs=[
            pl.BlockSpec((BS, N, 400), lambda i: (i, 0, 0)),
            pl.BlockSpec((BS, 26), lambda i: (i, 0)),
            fixed((400, HD)), fixed((1, HD)),
            fixed((HD, HD)), fixed((HD, H)), fixed((HD, BS * H)), fixed((1, HD)),
            fixed((HD, HD)), fixed((HD, H)), fixed((HD, BS * H)), fixed((1, HD)),
            fixed((HD, HD)), fixed((HD, H)), fixed((HD, BS * H)), fixed((1, HD)),
            fixed((HN, H)), fixed((H, HD)), fixed((H, HN)),
            fixed((SN, BS * H)), fixed((SN, HD)), fixed((BS, SN)),
            fixed((26, 128)), fixed((1, 128)),
            fixed((3 * HD + 128, 10)), fixed((1, 10)),
        ],
        out_specs=pl.BlockSpec((BS, 10), lambda i: (i, 0)),
        out_shape=jax.ShapeDtypeStruct((B, 10), F32),
    )(data, loading, W1.astype(BF16), b1.reshape(1, HD),
      fc1b, fcal1, fcar1, bg1.reshape(1, HD),
      fc2b, fcal2, fcar2, bg2.reshape(1, HD),
      fc3b, fcal3, fcar3, bg3.reshape(1, HD),
      msum, e32, rep8, colmask, rowmask, summat,
      Wl.astype(BF16), bl.reshape(1, 128),
      Wlast.astype(BF16), blast.reshape(1, 10))
    return out


# speedup vs baseline: 2.8857x; 1.0004x over previous
import jax
import jax.numpy as jnp
from jax.experimental import pallas as pl

N = 53
H = 8
D = 32
HD = H * D
NP = 64
HN = H * NP
BS = 32
SN = BS * NP

F32 = jnp.float32
BF16 = jnp.bfloat16


def _gat_body(data_ref, loading_ref, W1_ref, b1_ref,
              fc1_ref, fcal1_ref, fcar1_ref, bg1_ref,
              fc2_ref, fcal2_ref, fcar2_ref, bg2_ref,
              fc3_ref, fcal3_ref, fcar3_ref, bg3_ref,
              msum_ref, e32_ref, rep8_ref, colmask_ref, rowmask_ref,
              summat_ref,
              Wl_ref, bl_ref, Wlast_ref, blast_ref,
              out_ref):
    msum = msum_ref[...]
    e32 = e32_ref[...]
    rep8 = rep8_ref[...]
    colmask = colmask_ref[...]
    rowmask = rowmask_ref[...]

    hs_list = []
    for b in range(BS):
        x = data_ref[b]
        h_b = jnp.dot(x, W1_ref[...], preferred_element_type=F32) + b1_ref[...]
        hs_list.append(jnp.pad(jnp.maximum(h_b, 0.0), ((0, NP - N), (0, 0))))
    HS = jnp.concatenate(hs_list, axis=0)

    layers = ((fc1_ref, fcal1_ref, fcar1_ref, bg1_ref),
              (fc2_ref, fcal2_ref, fcar2_ref, bg2_ref),
              (fc3_ref, fcal3_ref, fcar3_ref, bg3_ref))
    feats = []
    for fc_ref, fcal_ref, fcart_ref, bg_ref in layers:
        HSb = HS.astype(BF16)
        FT = jnp.dot(HSb, fc_ref[...],
                     preferred_element_type=F32).astype(BF16)
        ERT = jnp.dot(HSb, fcart_ref[...],
                      preferred_element_type=F32)
        HB2 = jnp.concatenate(
            [HSb[NP * b:NP * (b + 1)] for b in range(BS) for _ in range(2)],
            axis=0)
        EL2 = jax.lax.dot_general(
            fcal_ref[...], HB2, (((0,), (1,)), ((), ())),
            preferred_element_type=F32)
        rp1s, rp2s = [], []
        for b in range(BS):
            el2b = EL2[:, 128 * b:128 * (b + 1)]
            rp1s.append(rep8 * jnp.tile(jnp.exp(el2b), (1, HN // 128)))
            rp2s.append(rep8 * jnp.tile(jnp.exp(0.2 * el2b), (1, HN // 128)))
        RP1 = jnp.concatenate(rp1s, axis=0).astype(BF16)
        RP2 = jnp.concatenate(rp2s, axis=0).astype(BF16)
        E1 = (colmask * jnp.exp(ERT)).astype(BF16)
        E2 = (colmask * jnp.exp(0.2 * ERT)).astype(BF16)
        U1 = jnp.dot(E1, RP1, preferred_element_type=F32)
        U2 = jnp.dot(E2, RP2, preferred_element_type=F32)
        EXB = jnp.maximum(U1, U2).astype(BF16)
        DEN = jnp.dot(EXB, msum, preferred_element_type=F32)
        SCREP = jnp.dot((1.0 / DEN).astype(BF16), e32,
                        preferred_element_type=F32)
        raws = []
        for b in range(BS):
            ftb = FT[NP * b:NP * (b + 1)]
            ftstack = jnp.concatenate(
                [ftb * e32[hd:hd + 1, :] for hd in range(H)], axis=0)
            raws.append(jnp.dot(EXB[NP * b:NP * (b + 1)], ftstack,
                                preferred_element_type=F32))
        RAW = jnp.concatenate(raws, axis=0)
        HS = rowmask * jnp.maximum(RAW * SCREP + HS + bg_ref[...], 0.0)
        feats.append(jnp.dot(summat_ref[...], HS.astype(BF16),
                             preferred_element_type=F32))

    lf = jnp.dot(loading_ref[...].astype(BF16), Wl_ref[...],
                 preferred_element_type=F32)
    lf = lf + bl_ref[...]
    lf = jnp.where(lf >= 0.0, lf, 0.01 * lf)
    lfb = lf.astype(BF16)

    f1 = feats[0].astype(BF16)
    f2 = feats[1].astype(BF16)
    f3 = feats[2].astype(BF16)
    o = jnp.dot(f1, Wlast_ref[0:HD, :], preferred_element_type=F32)
    o = o + jnp.dot(f2, Wlast_ref[HD:2 * HD, :], preferred_element_type=F32)
    o = o + jnp.dot(f3, Wlast_ref[2 * HD:3 * HD, :], preferred_element_type=F32)
    o = o + jnp.dot(lfb, Wlast_ref[3 * HD:3 * HD + 128, :],
                    preferred_element_type=F32)
    out_ref[...] = o + blast_ref[...]


def _block_diag_attn(a):
    mask = jnp.kron(jnp.eye(H, dtype=F32), jnp.ones((D, 1), dtype=F32))
    return mask * a.reshape(HD, 1)


def kernel(data, loading, edge_index, W1, b1, fcW1, al1, ar1, bg1,
           fcW2, al2, ar2, bg2, fcW3, al3, ar3, bg3, Wl, bl, Wlast, blast):
    B = data.shape[0]

    def prep_layer(fcW, al, ar):
        albd = _block_diag_attn(al)
        arbd = _block_diag_attn(ar)
        fcal = jnp.dot(fcW, albd)
        fcar = jnp.dot(fcW, arbd)
        fcart = jnp.tile(fcar, (1, BS))
        return fcW.astype(BF16), fcal.astype(BF16), fcart.astype(BF16)

    fc1b, fcal1, fcar1 = prep_layer(fcW1, al1, ar1)
    fc2b, fcal2, fcar2 = prep_layer(fcW2, al2, ar2)
    fc3b, fcal3, fcar3 = prep_layer(fcW3, al3, ar3)

    lane_i = jnp.arange(HN) % NP
    msum = jnp.kron(jnp.eye(H, dtype=F32), jnp.ones((NP, 1), dtype=F32))
    msum = (msum * (lane_i < N).astype(F32)[:, None]).astype(BF16)
    e32 = jnp.kron(jnp.eye(H, dtype=BF16), jnp.ones((1, D), dtype=BF16))
    rep8 = jnp.kron(jnp.eye(H, dtype=F32), jnp.ones((1, NP), dtype=F32))
    colmask = jnp.kron(jnp.eye(BS, dtype=F32), jnp.ones((NP, H), dtype=F32))
    rowvalid = (jnp.arange(SN) % NP < N).astype(F32)
    rowmask = jnp.tile(rowvalid[:, None], (1, HD))
    summat = jnp.kron(jnp.eye(BS, dtype=F32),
                      (jnp.arange(NP) < N).astype(F32)[None, :]).astype(BF16)

    def fixed(shape):
        nd = len(shape)
        return pl.BlockSpec(shape, lambda i: (0,) * nd)

    out = pl.pallas_call(
        _gat_body,
        grid=(B // BS,),
        in_specs=[
            pl.BlockSpec((BS, N, 400), lambda i: (i, 0, 0)),
            pl.BlockSpec((BS, 26), lambda i: (i, 0)),
            fixed((400, HD)), fixed((1, HD)),
            fixed((HD, HD)), fixed((HD, H)), fixed((HD, BS * H)), fixed((1, HD)),
            fixed((HD, HD)), fixed((HD, H)), fixed((HD, BS * H)), fixed((1, HD)),
            fixed((HD, HD)), fixed((HD, H)), fixed((HD, BS * H)), fixed((1, HD)),
            fixed((HN, H)), fixed((H, HD)), fixed((H, HN)),
            fixed((SN, BS * H)), fixed((SN, HD)), fixed((BS, SN)),
            fixed((26, 128)), fixed((1, 128)),
            fixed((3 * HD + 128, 10)), fixed((1, 10)),
        ],
        out_specs=pl.BlockSpec((BS, 10), lambda i: (i, 0)),
        out_shape=jax.ShapeDtypeStruct((B, 10), F32),
    )(data.astype(BF16), loading, W1.astype(BF16), b1.reshape(1, HD),
      fc1b, fcal1, fcar1, bg1.reshape(1, HD),
      fc2b, fcal2, fcar2, bg2.reshape(1, HD),
      fc3b, fcal3, fcar3, bg3.reshape(1, HD),
      msum, e32, rep8, colmask, rowmask, summat,
      Wl.astype(BF16), bl.reshape(1, 128),
      Wlast.astype(BF16), blast.reshape(1, 10))
    return out


# speedup vs baseline: 3.0392x; 1.0532x over previous
import jax
import jax.numpy as jnp
from jax.experimental import pallas as pl

N = 53
H = 8
D = 32
HD = H * D
NP = 64
HN = H * NP
BS = 32
GG = 8
NG = BS // GG
SN = BS * NP

F32 = jnp.float32
BF16 = jnp.bfloat16


def _gat_body(data_ref, loading_ref, W1_ref, b1_ref,
              fc1_ref, fcal1_ref, fcar1_ref, bg1_ref,
              fc2_ref, fcal2_ref, fcar2_ref, bg2_ref,
              fc3_ref, fcal3_ref, fcar3_ref, bg3_ref,
              msum_ref, e32_ref, rep8_ref, colmask_ref, rowmask_ref,
              summat_ref,
              Wl_ref, bl_ref, Wlast_ref, blast_ref,
              out_ref):
    msum = msum_ref[...]
    e32 = e32_ref[...]
    rep8 = rep8_ref[...]
    colmask = colmask_ref[...]
    rowmask = rowmask_ref[...]

    hs_list = []
    for b in range(BS):
        x = data_ref[b]
        h_b = jnp.dot(x, W1_ref[...], preferred_element_type=F32) + b1_ref[...]
        hs_list.append(jnp.pad(jnp.maximum(h_b, 0.0), ((0, NP - N), (0, 0))))
    HS = jnp.concatenate(hs_list, axis=0)

    layers = ((fc1_ref, fcal1_ref, fcar1_ref, bg1_ref),
              (fc2_ref, fcal2_ref, fcar2_ref, bg2_ref),
              (fc3_ref, fcal3_ref, fcar3_ref, bg3_ref))
    feats = []
    for fc_ref, fcal_ref, fcart_ref, bg_ref in layers:
        HSb = HS.astype(BF16)
        FT = jnp.dot(HSb, fc_ref[...],
                     preferred_element_type=F32).astype(BF16)
        ERT = jnp.dot(HSb, fcart_ref[...],
                      preferred_element_type=F32)
        HB2 = jnp.concatenate(
            [HSb[NP * b:NP * (b + 1)] for b in range(BS) for _ in range(2)],
            axis=0)
        EL2 = jax.lax.dot_general(
            fcal_ref[...], HB2, (((0,), (1,)), ((), ())),
            preferred_element_type=F32)
        rp1s, rp2s = [], []
        for b in range(BS):
            el2b = EL2[:, 128 * b:128 * (b + 1)]
            rp1s.append(rep8 * jnp.tile(jnp.exp(el2b), (1, HN // 128)))
            rp2s.append(rep8 * jnp.tile(jnp.exp(0.2 * el2b), (1, HN // 128)))
        RP1 = jnp.concatenate(rp1s, axis=0).astype(BF16)
        RP2 = jnp.concatenate(rp2s, axis=0).astype(BF16)
        E1 = (colmask * jnp.exp(ERT)).astype(BF16)
        E2 = (colmask * jnp.exp(0.2 * ERT)).astype(BF16)
        exbs = []
        for g in range(NG):
            rsl = slice(NP * GG * g, NP * GG * (g + 1))
            ksl = slice(H * GG * g, H * GG * (g + 1))
            U1 = jnp.dot(E1[rsl], RP1[ksl], preferred_element_type=F32)
            U2 = jnp.dot(E2[rsl], RP2[ksl], preferred_element_type=F32)
            exbs.append(jnp.maximum(U1, U2).astype(BF16))
        EXB = jnp.concatenate(exbs, axis=0)
        DEN = jnp.dot(EXB, msum, preferred_element_type=F32)
        SCREP = jnp.dot((1.0 / DEN).astype(BF16), e32,
                        preferred_element_type=F32)
        raws = []
        for b in range(BS):
            ftb = FT[NP * b:NP * (b + 1)]
            ftstack = jnp.concatenate(
                [ftb * e32[hd:hd + 1, :] for hd in range(H)], axis=0)
            raws.append(jnp.dot(EXB[NP * b:NP * (b + 1)], ftstack,
                                preferred_element_type=F32))
        RAW = jnp.concatenate(raws, axis=0)
        HS = rowmask * jnp.maximum(RAW * SCREP + HS + bg_ref[...], 0.0)
        feats.append(jnp.dot(summat_ref[...], HS.astype(BF16),
                             preferred_element_type=F32))

    lf = jnp.dot(loading_ref[...].astype(BF16), Wl_ref[...],
                 preferred_element_type=F32)
    lf = lf + bl_ref[...]
    lf = jnp.where(lf >= 0.0, lf, 0.01 * lf)
    lfb = lf.astype(BF16)

    f1 = feats[0].astype(BF16)
    f2 = feats[1].astype(BF16)
    f3 = feats[2].astype(BF16)
    o = jnp.dot(f1, Wlast_ref[0:HD, :], preferred_element_type=F32)
    o = o + jnp.dot(f2, Wlast_ref[HD:2 * HD, :], preferred_element_type=F32)
    o = o + jnp.dot(f3, Wlast_ref[2 * HD:3 * HD, :], preferred_element_type=F32)
    o = o + jnp.dot(lfb, Wlast_ref[3 * HD:3 * HD + 128, :],
                    preferred_element_type=F32)
    out_ref[...] = o + blast_ref[...]


def _block_diag_attn(a):
    mask = jnp.kron(jnp.eye(H, dtype=F32), jnp.ones((D, 1), dtype=F32))
    return mask * a.reshape(HD, 1)


def kernel(data, loading, edge_index, W1, b1, fcW1, al1, ar1, bg1,
           fcW2, al2, ar2, bg2, fcW3, al3, ar3, bg3, Wl, bl, Wlast, blast):
    B = data.shape[0]

    def prep_layer(fcW, al, ar):
        albd = _block_diag_attn(al)
        arbd = _block_diag_attn(ar)
        fcal = jnp.dot(fcW, albd)
        fcar = jnp.dot(fcW, arbd)
        fcart = jnp.tile(fcar, (1, GG))
        return fcW.astype(BF16), fcal.astype(BF16), fcart.astype(BF16)

    fc1b, fcal1, fcar1 = prep_layer(fcW1, al1, ar1)
    fc2b, fcal2, fcar2 = prep_layer(fcW2, al2, ar2)
    fc3b, fcal3, fcar3 = prep_layer(fcW3, al3, ar3)

    lane_i = jnp.arange(HN) % NP
    msum = jnp.kron(jnp.eye(H, dtype=F32), jnp.ones((NP, 1), dtype=F32))
    msum = (msum * (lane_i < N).astype(F32)[:, None]).astype(BF16)
    e32 = jnp.kron(jnp.eye(H, dtype=BF16), jnp.ones((1, D), dtype=BF16))
    rep8 = jnp.kron(jnp.eye(H, dtype=F32), jnp.ones((1, NP), dtype=F32))
    colmask = jnp.tile(
        jnp.kron(jnp.eye(GG, dtype=F32), jnp.ones((NP, H), dtype=F32)),
        (BS // GG, 1))
    rowvalid = (jnp.arange(SN) % NP < N).astype(F32)
    rowmask = jnp.tile(rowvalid[:, None], (1, HD))
    summat = jnp.kron(jnp.eye(BS, dtype=F32),
                      (jnp.arange(NP) < N).astype(F32)[None, :]).astype(BF16)

    def fixed(shape):
        nd = len(shape)
        return pl.BlockSpec(shape, lambda i: (0,) * nd)

    out = pl.pallas_call(
        _gat_body,
        grid=(B // BS,),
        in_specs=[
            pl.BlockSpec((BS, N, 400), lambda i: (i, 0, 0)),
            pl.BlockSpec((BS, 26), lambda i: (i, 0)),
            fixed((400, HD)), fixed((1, HD)),
            fixed((HD, HD)), fixed((HD, H)), fixed((HD, H * GG)), fixed((1, HD)),
            fixed((HD, HD)), fixed((HD, H)), fixed((HD, H * GG)), fixed((1, HD)),
            fixed((HD, HD)), fixed((HD, H)), fixed((HD, H * GG)), fixed((1, HD)),
            fixed((HN, H)), fixed((H, HD)), fixed((H, HN)),
            fixed((SN, H * GG)), fixed((SN, HD)), fixed((BS, SN)),
            fixed((26, 128)), fixed((1, 128)),
            fixed((3 * HD + 128, 10)), fixed((1, 10)),
        ],
        out_specs=pl.BlockSpec((BS, 10), lambda i: (i, 0)),
        out_shape=jax.ShapeDtypeStruct((B, 10), F32),
    )(data.astype(BF16), loading, W1.astype(BF16), b1.reshape(1, HD),
      fc1b, fcal1, fcar1, bg1.reshape(1, HD),
      fc2b, fcal2, fcar2, bg2.reshape(1, HD),
      fc3b, fcal3, fcar3, bg3.reshape(1, HD),
      msum, e32, rep8, colmask, rowmask, summat,
      Wl.astype(BF16), bl.reshape(1, 128),
      Wlast.astype(BF16), blast.reshape(1, 10))
    return out
